# Initial kernel scaffold; baseline (speedup 1.0000x reference)
import jax
import jax.numpy as jnp
from jax.experimental import pallas as pl


def kernel(x_nchw, wt, bias, gamma, beta):
    raise NotImplementedError("write your pallas kernel here")


# speedup vs baseline: 1.1486x; 1.1486x over previous
import functools

import numpy as np
import jax
import jax.numpy as jnp
from jax.experimental import pallas as pl
from jax.experimental.pallas import tpu as pltpu

_BN_EPS = 1e-5
_LANES = 128


def _tap_stack(xb, m_ref, H, W):
    HW = H * W
    pieces = []
    for ky in range(3):
        for kx in range(3):
            k = ky * 3 + kx
            shift = (-((ky - 1) * W + (kx - 1))) % HW
            shifted = jnp.roll(xb, shift, axis=1) if shift else xb
            if ky == 1 and kx == 1:
                pieces.append(shifted)
            else:
                pieces.append(shifted * m_ref[k:k + 1, :])
    return jnp.concatenate(pieces, axis=0)


def _conv_stats_kernel(x_ref, w_ref, m_ref, conv_ref, stats_ref, *, H, W, Cout):
    xb = x_ref[0].astype(jnp.bfloat16)
    taps = _tap_stack(xb, m_ref[...], H, W)
    conv = jnp.dot(w_ref[...], taps,
                   preferred_element_type=jnp.float32)
    conv_ref[0] = conv.astype(jnp.bfloat16)
    s = jnp.sum(conv, axis=1, keepdims=True)
    sq = jnp.sum(conv * conv, axis=1, keepdims=True)
    pad = jnp.zeros((Cout, _LANES - 2), jnp.float32)
    stats_ref[0] = jnp.concatenate([s, sq, pad], axis=1)


def _apply_kernel(conv_ref, x_ref, scale_ref, shift_ref, o_ref):
    conv = conv_ref[0].astype(jnp.float32)
    y = jnp.maximum(conv * scale_ref[...] + shift_ref[...], 0.0)
    x = x_ref[0]
    o_ref[0] = y + jnp.concatenate([x, x], axis=0)


def _boundary_masks(H, W):
    hh = np.arange(H * W) // W
    ww = np.arange(H * W) % W
    rows = []
    for ky in range(3):
        for kx in range(3):
            dy, dx = ky - 1, kx - 1
            ok = ((hh + dy >= 0) & (hh + dy <= H - 1) &
                  (ww + dx >= 0) & (ww + dx <= W - 1))
            rows.append(ok)
    return jnp.asarray(np.stack(rows, axis=0), dtype=jnp.bfloat16)


def kernel(x_nchw, wt, bias, gamma, beta):
    N, Cin, H, W = x_nchw.shape
    Cout = wt.shape[1]
    del bias
    f32 = jnp.float32
    HW = H * W

    x_planes = x_nchw.reshape(N, Cin, HW).astype(f32)

    w_flip = jnp.flip(wt, axis=(2, 3))
    w_stk = (jnp.transpose(w_flip, (2, 3, 0, 1))
             .reshape(9 * Cin, Cout).T.astype(jnp.bfloat16))

    masks = _boundary_masks(H, W)

    cparams = pltpu.CompilerParams(
        dimension_semantics=("parallel",),
        vmem_limit_bytes=100 << 20,
    )
    conv_flops = 2 * N * Cout * 9 * Cin * HW

    conv_b, stats = pl.pallas_call(
        functools.partial(_conv_stats_kernel, H=H, W=W, Cout=Cout),
        out_shape=(jax.ShapeDtypeStruct((N, Cout, HW), jnp.bfloat16),
                   jax.ShapeDtypeStruct((N, Cout, _LANES), f32)),
        grid_spec=pl.GridSpec(
            grid=(N,),
            in_specs=[
                pl.BlockSpec((1, Cin, HW), lambda n: (n, 0, 0)),
                pl.BlockSpec((Cout, 9 * Cin), lambda n: (0, 0)),
                pl.BlockSpec((9, HW), lambda n: (0, 0)),
            ],
            out_specs=(pl.BlockSpec((1, Cout, HW), lambda n: (n, 0, 0)),
                       pl.BlockSpec((1, Cout, _LANES), lambda n: (n, 0, 0))),
        ),
        compiler_params=cparams,
        cost_estimate=pl.CostEstimate(
            flops=conv_flops, transcendentals=0,
            bytes_accessed=4 * N * Cin * HW + 2 * N * Cout * HW
            + 4 * N * Cout * _LANES),
    )(x_planes, w_stk, masks)

    tot = jnp.sum(stats, axis=0)
    count = N * HW
    mean = tot[:, 0] / count
    var = tot[:, 1] / count - mean * mean
    inv_std = jax.lax.rsqrt(var + _BN_EPS)
    scale_c = gamma.astype(f32) * inv_std
    shift_c = beta.astype(f32) - mean * scale_c
    scale = scale_c.reshape(Cout, 1)
    shift = shift_c.reshape(Cout, 1)

    out = pl.pallas_call(
        _apply_kernel,
        out_shape=jax.ShapeDtypeStruct((N, Cout, HW), f32),
        grid_spec=pl.GridSpec(
            grid=(N,),
            in_specs=[
                pl.BlockSpec((1, Cout, HW), lambda n: (n, 0, 0)),
                pl.BlockSpec((1, Cin, HW), lambda n: (n, 0, 0)),
                pl.BlockSpec((Cout, 1), lambda n: (0, 0)),
                pl.BlockSpec((Cout, 1), lambda n: (0, 0)),
            ],
            out_specs=pl.BlockSpec((1, Cout, HW), lambda n: (n, 0, 0)),
        ),
        compiler_params=cparams,
        cost_estimate=pl.CostEstimate(
            flops=4 * N * Cout * HW, transcendentals=0,
            bytes_accessed=2 * N * Cout * HW + 4 * N * Cin * HW
            + 4 * N * Cout * HW),
    )(conv_b, x_planes, scale, shift)

    return out.reshape(N, Cout, H, W)


# speedup vs baseline: 1.5448x; 1.3449x over previous
import functools

import numpy as np
import jax
import jax.numpy as jnp
from jax.experimental import pallas as pl
from jax.experimental.pallas import tpu as pltpu

_BN_EPS = 1e-5
_LANES = 128


def _tap_stack(xb, m_ref, H, W):
    HW = H * W
    pieces = []
    for ky in range(3):
        for kx in range(3):
            k = ky * 3 + kx
            shift = (-((ky - 1) * W + (kx - 1))) % HW
            shifted = jnp.roll(xb, shift, axis=1) if shift else xb
            if ky == 1 and kx == 1:
                pieces.append(shifted)
            else:
                pieces.append(shifted * m_ref[k:k + 1, :])
    return jnp.concatenate(pieces, axis=0)


def _stats_kernel(x_ref, w_ref, m_ref, stats_ref, *, H, W, Cout):
    Cin = x_ref.shape[1]
    xb = x_ref[0].astype(jnp.bfloat16).reshape(Cin, H * W)
    taps = _tap_stack(xb, m_ref[...], H, W)
    conv = jnp.dot(w_ref[...], taps,
                   preferred_element_type=jnp.float32)
    s = jnp.sum(conv, axis=1, keepdims=True)
    sq = jnp.sum(conv * conv, axis=1, keepdims=True)
    pad = jnp.zeros((Cout, _LANES - 2), jnp.float32)
    stats_ref[0] = jnp.concatenate([s, sq, pad], axis=1)


def _apply_kernel(x_ref, w_ref, m_ref, scale_ref, shift_ref, o_ref, *, H, W):
    Cin = x_ref.shape[1]
    x = x_ref[0].reshape(Cin, H * W)
    xb = x.astype(jnp.bfloat16)
    taps = _tap_stack(xb, m_ref[...], H, W)
    conv = jnp.dot(w_ref[...], taps,
                   preferred_element_type=jnp.float32)
    y = jnp.maximum(conv * scale_ref[...] + shift_ref[...], 0.0)
    out = y + jnp.concatenate([x, x], axis=0)
    o_ref[0] = out.T


def _boundary_masks(H, W):
    hh = np.arange(H * W) // W
    ww = np.arange(H * W) % W
    rows = []
    for ky in range(3):
        for kx in range(3):
            dy, dx = ky - 1, kx - 1
            ok = ((hh + dy >= 0) & (hh + dy <= H - 1) &
                  (ww + dx >= 0) & (ww + dx <= W - 1))
            rows.append(ok)
    return jnp.asarray(np.stack(rows, axis=0), dtype=jnp.bfloat16)


def kernel(x_nchw, wt, bias, gamma, beta):
    N, Cin, H, W = x_nchw.shape
    Cout = wt.shape[1]
    del bias
    f32 = jnp.float32
    HW = H * W

    w_flip = jnp.flip(wt, axis=(2, 3))
    w_stk = (jnp.transpose(w_flip, (2, 3, 0, 1))
             .reshape(9 * Cin, Cout).T.astype(jnp.bfloat16))

    masks = _boundary_masks(H, W)

    cparams = pltpu.CompilerParams(
        dimension_semantics=("parallel",),
        vmem_limit_bytes=100 << 20,
    )
    conv_flops = 2 * N * Cout * 9 * Cin * HW

    stats = pl.pallas_call(
        functools.partial(_stats_kernel, H=H, W=W, Cout=Cout),
        out_shape=jax.ShapeDtypeStruct((N, Cout, _LANES), f32),
        grid_spec=pl.GridSpec(
            grid=(N,),
            in_specs=[
                pl.BlockSpec((1, Cin, H, W), lambda n: (n, 0, 0, 0)),
                pl.BlockSpec((Cout, 9 * Cin), lambda n: (0, 0)),
                pl.BlockSpec((9, HW), lambda n: (0, 0)),
            ],
            out_specs=pl.BlockSpec((1, Cout, _LANES), lambda n: (n, 0, 0)),
        ),
        compiler_params=cparams,
        cost_estimate=pl.CostEstimate(
            flops=conv_flops, transcendentals=0,
            bytes_accessed=4 * N * Cin * HW + 4 * N * Cout * _LANES),
    )(x_nchw, w_stk, masks)

    tot = jnp.sum(stats, axis=0)
    count = N * HW
    mean = tot[:, 0] / count
    var = tot[:, 1] / count - mean * mean
    inv_std = jax.lax.rsqrt(var + _BN_EPS)
    scale_c = gamma.astype(f32) * inv_std
    shift_c = beta.astype(f32) - mean * scale_c
    scale = scale_c.reshape(Cout, 1)
    shift = shift_c.reshape(Cout, 1)

    out_hwc = pl.pallas_call(
        functools.partial(_apply_kernel, H=H, W=W),
        out_shape=jax.ShapeDtypeStruct((N, HW, Cout), f32),
        grid_spec=pl.GridSpec(
            grid=(N,),
            in_specs=[
                pl.BlockSpec((1, Cin, H, W), lambda n: (n, 0, 0, 0)),
                pl.BlockSpec((Cout, 9 * Cin), lambda n: (0, 0)),
                pl.BlockSpec((9, HW), lambda n: (0, 0)),
                pl.BlockSpec((Cout, 1), lambda n: (0, 0)),
                pl.BlockSpec((Cout, 1), lambda n: (0, 0)),
            ],
            out_specs=pl.BlockSpec((1, HW, Cout), lambda n: (n, 0, 0)),
        ),
        compiler_params=cparams,
        cost_estimate=pl.CostEstimate(
            flops=conv_flops + 4 * N * Cout * HW, transcendentals=0,
            bytes_accessed=4 * N * Cin * HW + 4 * N * Cout * HW),
    )(x_nchw, w_stk, masks, scale, shift)

    return jnp.transpose(out_hwc.reshape(N, H, W, Cout), (0, 3, 1, 2))


# speedup vs baseline: 1.7676x; 1.1442x over previous
import functools

import numpy as np
import jax
import jax.numpy as jnp
from jax.experimental import pallas as pl
from jax.experimental.pallas import tpu as pltpu

_BN_EPS = 1e-5
_LANES = 128


def _tap_stack(xb, m_ref, H, W):
    HW = H * W
    pieces = []
    for ky in range(3):
        for kx in range(3):
            k = ky * 3 + kx
            shift = (-((ky - 1) * W + (kx - 1))) % HW
            shifted = jnp.roll(xb, shift, axis=1) if shift else xb
            if ky == 1 and kx == 1:
                pieces.append(shifted)
            else:
                pieces.append(shifted * m_ref[k:k + 1, :])
    return jnp.concatenate(pieces, axis=0)


def _stats_kernel(x_ref, w_ref, m_ref, stats_ref, xb_ref, *, H, W, Cout):
    Cin = x_ref.shape[1]
    xb = x_ref[0].astype(jnp.bfloat16).reshape(Cin, H * W)
    xb_ref[0] = xb
    taps = _tap_stack(xb, m_ref[...], H, W)
    conv = jnp.dot(w_ref[...], taps,
                   preferred_element_type=jnp.float32)
    s = jnp.sum(conv, axis=1, keepdims=True)
    sq = jnp.sum(conv * conv, axis=1, keepdims=True)
    pad = jnp.zeros((Cout, _LANES - 2), jnp.float32)
    stats_ref[0] = jnp.concatenate([s, sq, pad], axis=1)


def _apply_kernel(xb_ref, w_ref, m_ref, scale_ref, shift_ref, o_ref, *, H, W):
    xb = xb_ref[0]
    taps = _tap_stack(xb, m_ref[...], H, W)
    conv = jnp.dot(w_ref[...], taps,
                   preferred_element_type=jnp.float32)
    y = jnp.maximum(conv * scale_ref[...] + shift_ref[...], 0.0)
    x = xb.astype(jnp.float32)
    out = y + jnp.concatenate([x, x], axis=0)
    o_ref[0] = out.T


def _boundary_masks(H, W):
    hh = np.arange(H * W) // W
    ww = np.arange(H * W) % W
    rows = []
    for ky in range(3):
        for kx in range(3):
            dy, dx = ky - 1, kx - 1
            ok = ((hh + dy >= 0) & (hh + dy <= H - 1) &
                  (ww + dx >= 0) & (ww + dx <= W - 1))
            rows.append(ok)
    return jnp.asarray(np.stack(rows, axis=0), dtype=jnp.bfloat16)


def kernel(x_nchw, wt, bias, gamma, beta):
    N, Cin, H, W = x_nchw.shape
    Cout = wt.shape[1]
    del bias
    f32 = jnp.float32
    HW = H * W

    w_flip = jnp.flip(wt, axis=(2, 3))
    w_stk = (jnp.transpose(w_flip, (2, 3, 0, 1))
             .reshape(9 * Cin, Cout).T.astype(jnp.bfloat16))

    masks = _boundary_masks(H, W)

    cparams = pltpu.CompilerParams(
        dimension_semantics=("parallel",),
        vmem_limit_bytes=100 << 20,
    )
    conv_flops = 2 * N * Cout * 9 * Cin * HW

    stats, xb_planes = pl.pallas_call(
        functools.partial(_stats_kernel, H=H, W=W, Cout=Cout),
        out_shape=(jax.ShapeDtypeStruct((N, Cout, _LANES), f32),
                   jax.ShapeDtypeStruct((N, Cin, HW), jnp.bfloat16)),
        grid_spec=pl.GridSpec(
            grid=(N,),
            in_specs=[
                pl.BlockSpec((1, Cin, H, W), lambda n: (n, 0, 0, 0)),
                pl.BlockSpec((Cout, 9 * Cin), lambda n: (0, 0)),
                pl.BlockSpec((9, HW), lambda n: (0, 0)),
            ],
            out_specs=(pl.BlockSpec((1, Cout, _LANES), lambda n: (n, 0, 0)),
                       pl.BlockSpec((1, Cin, HW), lambda n: (n, 0, 0))),
        ),
        compiler_params=cparams,
        cost_estimate=pl.CostEstimate(
            flops=conv_flops, transcendentals=0,
            bytes_accessed=4 * N * Cin * HW + 2 * N * Cin * HW
            + 4 * N * Cout * _LANES),
    )(x_nchw, w_stk, masks)

    tot = jnp.sum(stats, axis=0)
    count = N * HW
    mean = tot[:, 0] / count
    var = tot[:, 1] / count - mean * mean
    inv_std = jax.lax.rsqrt(var + _BN_EPS)
    scale_c = gamma.astype(f32) * inv_std
    shift_c = beta.astype(f32) - mean * scale_c
    scale = scale_c.reshape(Cout, 1)
    shift = shift_c.reshape(Cout, 1)

    out_hwc = pl.pallas_call(
        functools.partial(_apply_kernel, H=H, W=W),
        out_shape=jax.ShapeDtypeStruct((N, HW, Cout), f32),
        grid_spec=pl.GridSpec(
            grid=(N,),
            in_specs=[
                pl.BlockSpec((1, Cin, HW), lambda n: (n, 0, 0)),
                pl.BlockSpec((Cout, 9 * Cin), lambda n: (0, 0)),
                pl.BlockSpec((9, HW), lambda n: (0, 0)),
                pl.BlockSpec((Cout, 1), lambda n: (0, 0)),
                pl.BlockSpec((Cout, 1), lambda n: (0, 0)),
            ],
            out_specs=pl.BlockSpec((1, HW, Cout), lambda n: (n, 0, 0)),
        ),
        compiler_params=cparams,
        cost_estimate=pl.CostEstimate(
            flops=conv_flops + 4 * N * Cout * HW, transcendentals=0,
            bytes_accessed=2 * N * Cin * HW + 4 * N * Cout * HW),
    )(xb_planes, w_stk, masks, scale, shift)

    return jnp.transpose(out_hwc.reshape(N, H, W, Cout), (0, 3, 1, 2))


# speedup vs baseline: 1.9070x; 1.0788x over previous
import functools

import numpy as np
import jax
import jax.numpy as jnp
from jax.experimental import pallas as pl
from jax.experimental.pallas import tpu as pltpu

_BN_EPS = 1e-5
_LANES = 128


def _tap_stack(xb, m_ref, H, W):
    HW = H * W
    pieces = []
    for ky in range(3):
        for kx in range(3):
            k = ky * 3 + kx
            shift = (-((ky - 1) * W + (kx - 1))) % HW
            shifted = jnp.roll(xb, shift, axis=1) if shift else xb
            if ky == 1 and kx == 1:
                pieces.append(shifted)
            else:
                pieces.append(shifted * m_ref[k:k + 1, :])
    return jnp.concatenate(pieces, axis=0)


def _stats_kernel(x_ref, w_ref, m_ref, stats_ref, xb_ref, *, H, W, Cout):
    Cin = x_ref.shape[1]
    for i in range(x_ref.shape[0]):
        xb = x_ref[i].astype(jnp.bfloat16).reshape(Cin, H * W)
        xb_ref[i] = xb
        taps = _tap_stack(xb, m_ref[...], H, W)
        conv = jnp.dot(w_ref[...], taps,
                       preferred_element_type=jnp.float32)
        s = jnp.sum(conv, axis=1, keepdims=True)
        sq = jnp.sum(conv * conv, axis=1, keepdims=True)
        pad = jnp.zeros((Cout, _LANES - 2), jnp.float32)
        stats_ref[i] = jnp.concatenate([s, sq, pad], axis=1)


def _apply_kernel(xb_ref, w_ref, m_ref, scale_ref, shift_ref, o_ref, *, H, W):
    for i in range(xb_ref.shape[0]):
        xb = xb_ref[i]
        taps = _tap_stack(xb, m_ref[...], H, W)
        conv = jnp.dot(w_ref[...], taps,
                       preferred_element_type=jnp.float32)
        y = jnp.maximum(conv * scale_ref[...] + shift_ref[...], 0.0)
        x = xb.astype(jnp.float32)
        out = y + jnp.concatenate([x, x], axis=0)
        o_ref[i] = out.T


def _boundary_masks(H, W):
    hh = np.arange(H * W) // W
    ww = np.arange(H * W) % W
    rows = []
    for ky in range(3):
        for kx in range(3):
            dy, dx = ky - 1, kx - 1
            ok = ((hh + dy >= 0) & (hh + dy <= H - 1) &
                  (ww + dx >= 0) & (ww + dx <= W - 1))
            rows.append(ok)
    return jnp.asarray(np.stack(rows, axis=0), dtype=jnp.bfloat16)


def kernel(x_nchw, wt, bias, gamma, beta):
    N, Cin, H, W = x_nchw.shape
    Cout = wt.shape[1]
    del bias
    f32 = jnp.float32
    HW = H * W

    w_flip = jnp.flip(wt, axis=(2, 3))
    w_stk = (jnp.transpose(w_flip, (2, 3, 0, 1))
             .reshape(9 * Cin, Cout).T.astype(jnp.bfloat16))

    masks = _boundary_masks(H, W)

    cparams = pltpu.CompilerParams(
        dimension_semantics=("parallel",),
        vmem_limit_bytes=100 << 20,
    )
    conv_flops = 2 * N * Cout * 9 * Cin * HW

    nb = 2
    stats, xb_planes = pl.pallas_call(
        functools.partial(_stats_kernel, H=H, W=W, Cout=Cout),
        out_shape=(jax.ShapeDtypeStruct((N, Cout, _LANES), f32),
                   jax.ShapeDtypeStruct((N, Cin, HW), jnp.bfloat16)),
        grid_spec=pl.GridSpec(
            grid=(N // nb,),
            in_specs=[
                pl.BlockSpec((nb, Cin, H, W), lambda n: (n, 0, 0, 0)),
                pl.BlockSpec((Cout, 9 * Cin), lambda n: (0, 0)),
                pl.BlockSpec((9, HW), lambda n: (0, 0)),
            ],
            out_specs=(pl.BlockSpec((nb, Cout, _LANES), lambda n: (n, 0, 0)),
                       pl.BlockSpec((nb, Cin, HW), lambda n: (n, 0, 0))),
        ),
        compiler_params=cparams,
        cost_estimate=pl.CostEstimate(
            flops=conv_flops, transcendentals=0,
            bytes_accessed=4 * N * Cin * HW + 2 * N * Cin * HW
            + 4 * N * Cout * _LANES),
    )(x_nchw, w_stk, masks)

    tot = jnp.sum(stats, axis=0)
    count = N * HW
    mean = tot[:, 0] / count
    var = tot[:, 1] / count - mean * mean
    inv_std = jax.lax.rsqrt(var + _BN_EPS)
    scale_c = gamma.astype(f32) * inv_std
    shift_c = beta.astype(f32) - mean * scale_c
    scale = scale_c.reshape(Cout, 1)
    shift = shift_c.reshape(Cout, 1)

    out_hwc = pl.pallas_call(
        functools.partial(_apply_kernel, H=H, W=W),
        out_shape=jax.ShapeDtypeStruct((N, HW, Cout), f32),
        grid_spec=pl.GridSpec(
            grid=(N // nb,),
            in_specs=[
                pl.BlockSpec((nb, Cin, HW), lambda n: (n, 0, 0)),
                pl.BlockSpec((Cout, 9 * Cin), lambda n: (0, 0)),
                pl.BlockSpec((9, HW), lambda n: (0, 0)),
                pl.BlockSpec((Cout, 1), lambda n: (0, 0)),
                pl.BlockSpec((Cout, 1), lambda n: (0, 0)),
            ],
            out_specs=pl.BlockSpec((nb, HW, Cout), lambda n: (n, 0, 0)),
        ),
        compiler_params=cparams,
        cost_estimate=pl.CostEstimate(
            flops=conv_flops + 4 * N * Cout * HW, transcendentals=0,
            bytes_accessed=2 * N * Cin * HW + 4 * N * Cout * HW),
    )(xb_planes, w_stk, masks, scale, shift)

    return jnp.transpose(out_hwc.reshape(N, H, W, Cout), (0, 3, 1, 2))


# speedup vs baseline: 1.9506x; 1.0229x over previous
import functools

import numpy as np
import jax
import jax.numpy as jnp
from jax.experimental import pallas as pl
from jax.experimental.pallas import tpu as pltpu

_BN_EPS = 1e-5
_LANES = 128


def _tap_stack(xb, m_ref, H, W):
    HW = H * W
    pieces = []
    for ky in range(3):
        for kx in range(3):
            k = ky * 3 + kx
            shift = (-((ky - 1) * W + (kx - 1))) % HW
            shifted = jnp.roll(xb, shift, axis=1) if shift else xb
            if ky == 1 and kx == 1:
                pieces.append(shifted)
            else:
                pieces.append(shifted * m_ref[k:k + 1, :])
    return jnp.concatenate(pieces, axis=0)


def _stats_kernel(x_ref, w_ref, m_ref, stats_ref, xb_ref, *, H, W, Cout):
    Cin = x_ref.shape[1]
    for i in range(x_ref.shape[0]):
        xb = x_ref[i].astype(jnp.bfloat16).reshape(Cin, H * W)
        xb_ref[i] = xb
        taps = _tap_stack(xb, m_ref[...], H, W)
        conv = jnp.dot(w_ref[...], taps,
                       preferred_element_type=jnp.float32)
        s = jnp.sum(conv, axis=1, keepdims=True)
        sq = jnp.sum(conv * conv, axis=1, keepdims=True)
        pad = jnp.zeros((Cout, _LANES - 2), jnp.float32)
        stats_ref[i] = jnp.concatenate([s, sq, pad], axis=1)


def _apply_kernel(xb_ref, w_ref, m_ref, scale_ref, shift_ref, o_ref, *, H, W):
    for i in range(xb_ref.shape[0]):
        xb = xb_ref[i]
        taps = _tap_stack(xb, m_ref[...], H, W)
        conv = jnp.dot(w_ref[...], taps,
                       preferred_element_type=jnp.float32)
        y = jnp.maximum(conv * scale_ref[...] + shift_ref[...], 0.0)
        x = xb.astype(jnp.float32)
        out = y + jnp.concatenate([x, x], axis=0)
        o_ref[i] = out.T


def _boundary_masks(H, W):
    hh = np.arange(H * W) // W
    ww = np.arange(H * W) % W
    rows = []
    for ky in range(3):
        for kx in range(3):
            dy, dx = ky - 1, kx - 1
            ok = ((hh + dy >= 0) & (hh + dy <= H - 1) &
                  (ww + dx >= 0) & (ww + dx <= W - 1))
            rows.append(ok)
    return jnp.asarray(np.stack(rows, axis=0), dtype=jnp.bfloat16)


def kernel(x_nchw, wt, bias, gamma, beta):
    N, Cin, H, W = x_nchw.shape
    Cout = wt.shape[1]
    del bias
    f32 = jnp.float32
    HW = H * W

    w_flip = jnp.flip(wt, axis=(2, 3))
    w_stk = (jnp.transpose(w_flip, (2, 3, 0, 1))
             .reshape(9 * Cin, Cout).T.astype(jnp.bfloat16))

    masks = _boundary_masks(H, W)

    cparams = pltpu.CompilerParams(
        dimension_semantics=("parallel",),
        vmem_limit_bytes=100 << 20,
    )
    conv_flops = 2 * N * Cout * 9 * Cin * HW

    nb = 4
    stats, xb_planes = pl.pallas_call(
        functools.partial(_stats_kernel, H=H, W=W, Cout=Cout),
        out_shape=(jax.ShapeDtypeStruct((N, Cout, _LANES), f32),
                   jax.ShapeDtypeStruct((N, Cin, HW), jnp.bfloat16)),
        grid_spec=pl.GridSpec(
            grid=(N // nb,),
            in_specs=[
                pl.BlockSpec((nb, Cin, H, W), lambda n: (n, 0, 0, 0)),
                pl.BlockSpec((Cout, 9 * Cin), lambda n: (0, 0)),
                pl.BlockSpec((9, HW), lambda n: (0, 0)),
            ],
            out_specs=(pl.BlockSpec((nb, Cout, _LANES), lambda n: (n, 0, 0)),
                       pl.BlockSpec((nb, Cin, HW), lambda n: (n, 0, 0))),
        ),
        compiler_params=cparams,
        cost_estimate=pl.CostEstimate(
            flops=conv_flops, transcendentals=0,
            bytes_accessed=4 * N * Cin * HW + 2 * N * Cin * HW
            + 4 * N * Cout * _LANES),
    )(x_nchw, w_stk, masks)

    tot = jnp.sum(stats, axis=0)
    count = N * HW
    mean = tot[:, 0] / count
    var = tot[:, 1] / count - mean * mean
    inv_std = jax.lax.rsqrt(var + _BN_EPS)
    scale_c = gamma.astype(f32) * inv_std
    shift_c = beta.astype(f32) - mean * scale_c
    scale = scale_c.reshape(Cout, 1)
    shift = shift_c.reshape(Cout, 1)

    out_hwc = pl.pallas_call(
        functools.partial(_apply_kernel, H=H, W=W),
        out_shape=jax.ShapeDtypeStruct((N, HW, Cout), f32),
        grid_spec=pl.GridSpec(
            grid=(N // nb,),
            in_specs=[
                pl.BlockSpec((nb, Cin, HW), lambda n: (n, 0, 0)),
                pl.BlockSpec((Cout, 9 * Cin), lambda n: (0, 0)),
                pl.BlockSpec((9, HW), lambda n: (0, 0)),
                pl.BlockSpec((Cout, 1), lambda n: (0, 0)),
                pl.BlockSpec((Cout, 1), lambda n: (0, 0)),
            ],
            out_specs=pl.BlockSpec((nb, HW, Cout), lambda n: (n, 0, 0)),
        ),
        compiler_params=cparams,
        cost_estimate=pl.CostEstimate(
            flops=conv_flops + 4 * N * Cout * HW, transcendentals=0,
            bytes_accessed=2 * N * Cin * HW + 4 * N * Cout * HW),
    )(xb_planes, w_stk, masks, scale, shift)

    return jnp.transpose(out_hwc.reshape(N, H, W, Cout), (0, 3, 1, 2))


# speedup vs baseline: 2.0288x; 1.0400x over previous
import functools

import numpy as np
import jax
import jax.numpy as jnp
from jax.experimental import pallas as pl
from jax.experimental.pallas import tpu as pltpu

_BN_EPS = 1e-5
_LANES = 128


def _tap_stack(xb, m_ref, H, W):
    HW = H * W
    pieces = []
    for ky in range(3):
        for kx in range(3):
            k = ky * 3 + kx
            shift = (-((ky - 1) * W + (kx - 1))) % HW
            shifted = jnp.roll(xb, shift, axis=1) if shift else xb
            if ky == 1 and kx == 1:
                pieces.append(shifted)
            else:
                pieces.append(shifted * m_ref[k:k + 1, :])
    return jnp.concatenate(pieces, axis=0)


def _stats_kernel(x_ref, w_ref, m_ref, stats_ref, xb_ref, *, H, W, Cout):
    Cin = x_ref.shape[1]
    for i in range(x_ref.shape[0]):
        xb = x_ref[i].astype(jnp.bfloat16).reshape(Cin, H * W)
        xb_ref[i] = xb
        taps = _tap_stack(xb, m_ref[...], H, W)
        conv = jnp.dot(w_ref[...], taps,
                       preferred_element_type=jnp.float32)
        s = jnp.sum(conv, axis=1, keepdims=True)
        sq = jnp.sum(conv * conv, axis=1, keepdims=True)
        pad = jnp.zeros((Cout, _LANES - 2), jnp.float32)
        stats_ref[i] = jnp.concatenate([s, sq, pad], axis=1)


def _apply_kernel(xb_ref, w_ref, m_ref, scale_ref, shift_ref, o_ref, *, H, W):
    for i in range(xb_ref.shape[0]):
        xb = xb_ref[i]
        taps = _tap_stack(xb, m_ref[...], H, W)
        conv = jnp.dot(w_ref[...], taps,
                       preferred_element_type=jnp.float32)
        y = jnp.maximum(conv * scale_ref[...] + shift_ref[...], 0.0)
        x = xb.astype(jnp.float32)
        out = y + jnp.concatenate([x, x], axis=0)
        o_ref[i] = out.T


def _boundary_masks(H, W):
    hh = np.arange(H * W) // W
    ww = np.arange(H * W) % W
    rows = []
    for ky in range(3):
        for kx in range(3):
            dy, dx = ky - 1, kx - 1
            ok = ((hh + dy >= 0) & (hh + dy <= H - 1) &
                  (ww + dx >= 0) & (ww + dx <= W - 1))
            rows.append(ok)
    return jnp.asarray(np.stack(rows, axis=0), dtype=jnp.bfloat16)


def kernel(x_nchw, wt, bias, gamma, beta):
    N, Cin, H, W = x_nchw.shape
    Cout = wt.shape[1]
    del bias
    f32 = jnp.float32
    HW = H * W

    w_flip = jnp.flip(wt, axis=(2, 3))
    w_stk = (jnp.transpose(w_flip, (2, 3, 0, 1))
             .reshape(9 * Cin, Cout).T.astype(jnp.bfloat16))

    masks = _boundary_masks(H, W)

    cparams = pltpu.CompilerParams(
        dimension_semantics=("parallel",),
        vmem_limit_bytes=48 << 20,
    )
    conv_flops = 2 * N * Cout * 9 * Cin * HW

    nb = 4
    stats, xb_planes = pl.pallas_call(
        functools.partial(_stats_kernel, H=H, W=W, Cout=Cout),
        out_shape=(jax.ShapeDtypeStruct((N, Cout, _LANES), f32),
                   jax.ShapeDtypeStruct((N, Cin, HW), jnp.bfloat16)),
        grid_spec=pl.GridSpec(
            grid=(N // nb,),
            in_specs=[
                pl.BlockSpec((nb, Cin, H, W), lambda n: (n, 0, 0, 0)),
                pl.BlockSpec((Cout, 9 * Cin), lambda n: (0, 0)),
                pl.BlockSpec((9, HW), lambda n: (0, 0)),
            ],
            out_specs=(pl.BlockSpec((nb, Cout, _LANES), lambda n: (n, 0, 0)),
                       pl.BlockSpec((nb, Cin, HW), lambda n: (n, 0, 0))),
        ),
        compiler_params=cparams,
        cost_estimate=pl.CostEstimate(
            flops=conv_flops, transcendentals=0,
            bytes_accessed=4 * N * Cin * HW + 2 * N * Cin * HW
            + 4 * N * Cout * _LANES),
    )(x_nchw, w_stk, masks)

    tot = jnp.sum(stats, axis=0)
    count = N * HW
    mean = tot[:, 0] / count
    var = tot[:, 1] / count - mean * mean
    inv_std = jax.lax.rsqrt(var + _BN_EPS)
    scale_c = gamma.astype(f32) * inv_std
    shift_c = beta.astype(f32) - mean * scale_c
    scale = scale_c.reshape(Cout, 1)
    shift = shift_c.reshape(Cout, 1)

    out_hwc = pl.pallas_call(
        functools.partial(_apply_kernel, H=H, W=W),
        out_shape=jax.ShapeDtypeStruct((N, HW, Cout), f32),
        grid_spec=pl.GridSpec(
            grid=(N // nb,),
            in_specs=[
                pl.BlockSpec((nb, Cin, HW), lambda n: (n, 0, 0)),
                pl.BlockSpec((Cout, 9 * Cin), lambda n: (0, 0)),
                pl.BlockSpec((9, HW), lambda n: (0, 0)),
                pl.BlockSpec((Cout, 1), lambda n: (0, 0)),
                pl.BlockSpec((Cout, 1), lambda n: (0, 0)),
            ],
            out_specs=pl.BlockSpec((nb, HW, Cout), lambda n: (n, 0, 0)),
        ),
        compiler_params=cparams,
        cost_estimate=pl.CostEstimate(
            flops=conv_flops + 4 * N * Cout * HW, transcendentals=0,
            bytes_accessed=2 * N * Cin * HW + 4 * N * Cout * HW),
    )(xb_planes, w_stk, masks, scale, shift)

    return jnp.transpose(out_hwc.reshape(N, H, W, Cout), (0, 3, 1, 2))


# speedup vs baseline: 2.0472x; 1.0091x over previous
import functools

import numpy as np
import jax
import jax.numpy as jnp
from jax.experimental import pallas as pl
from jax.experimental.pallas import tpu as pltpu

_BN_EPS = 1e-5
_LANES = 128


def _tap_stack(xb, m_ref, H, W):
    HW = H * W
    pieces = []
    for ky in range(3):
        for kx in range(3):
            k = ky * 3 + kx
            shift = (-((ky - 1) * W + (kx - 1))) % HW
            shifted = jnp.roll(xb, shift, axis=1) if shift else xb
            if ky == 1 and kx == 1:
                pieces.append(shifted)
            else:
                pieces.append(shifted * m_ref[k:k + 1, :])
    return jnp.concatenate(pieces, axis=0)


def _stats_kernel(x_ref, w_ref, m_ref, stats_ref, xb_ref, *, H, W, Cout):
    Cin = x_ref.shape[1]
    for i in range(x_ref.shape[0]):
        xb = x_ref[i].astype(jnp.bfloat16).reshape(Cin, H * W)
        xb_ref[i] = xb
        taps = _tap_stack(xb, m_ref[...], H, W)
        conv = jnp.dot(w_ref[...], taps,
                       preferred_element_type=jnp.float32)
        s = jnp.sum(conv, axis=1, keepdims=True)
        sq = jnp.sum(conv * conv, axis=1, keepdims=True)
        pad = jnp.zeros((Cout, _LANES - 2), jnp.float32)
        stats_ref[i] = jnp.concatenate([s, sq, pad], axis=1)


def _apply_kernel(xb_ref, w_ref, m_ref, scale_ref, shift_ref, o_ref, *, H, W):
    for i in range(xb_ref.shape[0]):
        xb = xb_ref[i]
        taps = _tap_stack(xb, m_ref[...], H, W)
        conv = jnp.dot(w_ref[...], taps,
                       preferred_element_type=jnp.float32)
        y = jnp.maximum(conv * scale_ref[...] + shift_ref[...], 0.0)
        x = xb.astype(jnp.float32)
        out = y + jnp.concatenate([x, x], axis=0)
        o_ref[i] = out.T


def _boundary_masks(H, W):
    hh = np.arange(H * W) // W
    ww = np.arange(H * W) % W
    rows = []
    for ky in range(3):
        for kx in range(3):
            dy, dx = ky - 1, kx - 1
            ok = ((hh + dy >= 0) & (hh + dy <= H - 1) &
                  (ww + dx >= 0) & (ww + dx <= W - 1))
            rows.append(ok)
    return jnp.asarray(np.stack(rows, axis=0), dtype=jnp.bfloat16)


def kernel(x_nchw, wt, bias, gamma, beta):
    N, Cin, H, W = x_nchw.shape
    Cout = wt.shape[1]
    del bias
    f32 = jnp.float32
    HW = H * W

    w_flip = jnp.flip(wt, axis=(2, 3))
    w_stk = (jnp.transpose(w_flip, (2, 3, 0, 1))
             .reshape(9 * Cin, Cout).T.astype(jnp.bfloat16))

    masks = _boundary_masks(H, W)

    cparams = pltpu.CompilerParams(
        dimension_semantics=("parallel",),
        vmem_limit_bytes=48 << 20,
    )
    conv_flops = 2 * N * Cout * 9 * Cin * HW

    nb1, nb2 = 8, 4
    stats, xb_planes = pl.pallas_call(
        functools.partial(_stats_kernel, H=H, W=W, Cout=Cout),
        out_shape=(jax.ShapeDtypeStruct((N, Cout, _LANES), f32),
                   jax.ShapeDtypeStruct((N, Cin, HW), jnp.bfloat16)),
        grid_spec=pl.GridSpec(
            grid=(N // nb1,),
            in_specs=[
                pl.BlockSpec((nb1, Cin, H, W), lambda n: (n, 0, 0, 0)),
                pl.BlockSpec((Cout, 9 * Cin), lambda n: (0, 0)),
                pl.BlockSpec((9, HW), lambda n: (0, 0)),
            ],
            out_specs=(pl.BlockSpec((nb1, Cout, _LANES), lambda n: (n, 0, 0)),
                       pl.BlockSpec((nb1, Cin, HW), lambda n: (n, 0, 0))),
        ),
        compiler_params=cparams,
        cost_estimate=pl.CostEstimate(
            flops=conv_flops, transcendentals=0,
            bytes_accessed=4 * N * Cin * HW + 2 * N * Cin * HW
            + 4 * N * Cout * _LANES),
    )(x_nchw, w_stk, masks)

    tot = jnp.sum(stats, axis=0)
    count = N * HW
    mean = tot[:, 0] / count
    var = tot[:, 1] / count - mean * mean
    inv_std = jax.lax.rsqrt(var + _BN_EPS)
    scale_c = gamma.astype(f32) * inv_std
    shift_c = beta.astype(f32) - mean * scale_c
    scale = scale_c.reshape(Cout, 1)
    shift = shift_c.reshape(Cout, 1)

    out_hwc = pl.pallas_call(
        functools.partial(_apply_kernel, H=H, W=W),
        out_shape=jax.ShapeDtypeStruct((N, HW, Cout), f32),
        grid_spec=pl.GridSpec(
            grid=(N // nb2,),
            in_specs=[
                pl.BlockSpec((nb2, Cin, HW), lambda n: (n, 0, 0)),
                pl.BlockSpec((Cout, 9 * Cin), lambda n: (0, 0)),
                pl.BlockSpec((9, HW), lambda n: (0, 0)),
                pl.BlockSpec((Cout, 1), lambda n: (0, 0)),
                pl.BlockSpec((Cout, 1), lambda n: (0, 0)),
            ],
            out_specs=pl.BlockSpec((nb2, HW, Cout), lambda n: (n, 0, 0)),
        ),
        compiler_params=cparams,
        cost_estimate=pl.CostEstimate(
            flops=conv_flops + 4 * N * Cout * HW, transcendentals=0,
            bytes_accessed=2 * N * Cin * HW + 4 * N * Cout * HW),
    )(xb_planes, w_stk, masks, scale, shift)

    return jnp.transpose(out_hwc.reshape(N, H, W, Cout), (0, 3, 1, 2))


# speedup vs baseline: 2.1018x; 1.0267x over previous
import functools

import numpy as np
import jax
import jax.numpy as jnp
from jax.experimental import pallas as pl
from jax.experimental.pallas import tpu as pltpu

_BN_EPS = 1e-5
_LANES = 128


def _tap_stack(xb, m_ref, H, W):
    HW = H * W
    pieces = []
    for ky in range(3):
        for kx in range(3):
            k = ky * 3 + kx
            shift = (-((ky - 1) * W + (kx - 1))) % HW
            shifted = jnp.roll(xb, shift, axis=1) if shift else xb
            if ky == 1 and kx == 1:
                pieces.append(shifted)
            else:
                pieces.append(shifted * m_ref[k:k + 1, :])
    return jnp.concatenate(pieces, axis=0)


def _stats_kernel(x_ref, w_ref, m_ref, stats_ref, xb_ref, *, H, W, Cout):
    Cin = x_ref.shape[1]
    for i in range(x_ref.shape[0]):
        xb = x_ref[i].astype(jnp.bfloat16).reshape(Cin, H * W)
        xb_ref[i] = xb
        taps = _tap_stack(xb, m_ref[...], H, W)
        conv = jnp.dot(w_ref[...], taps,
                       preferred_element_type=jnp.float32)
        s = jnp.sum(conv, axis=1, keepdims=True)
        sq = jnp.sum(conv * conv, axis=1, keepdims=True)
        pad = jnp.zeros((Cout, _LANES - 2), jnp.float32)
        stats_ref[i] = jnp.concatenate([s, sq, pad], axis=1)


def _apply_kernel(xb_ref, w_ref, m_ref, o_ref, *, H, W):
    for i in range(xb_ref.shape[0]):
        xb = xb_ref[i]
        taps = _tap_stack(xb, m_ref[...], H, W)
        taps = jnp.concatenate([taps, m_ref[9:10, :]], axis=0)
        y = jnp.maximum(jnp.dot(w_ref[...], taps,
                                preferred_element_type=jnp.float32), 0.0)
        x = xb.astype(jnp.float32)
        out = y + jnp.concatenate([x, x], axis=0)
        o_ref[i] = out.T


def _boundary_masks(H, W):
    hh = np.arange(H * W) // W
    ww = np.arange(H * W) % W
    rows = []
    for ky in range(3):
        for kx in range(3):
            dy, dx = ky - 1, kx - 1
            ok = ((hh + dy >= 0) & (hh + dy <= H - 1) &
                  (ww + dx >= 0) & (ww + dx <= W - 1))
            rows.append(ok)
    rows.append(np.ones(H * W, dtype=bool))
    return jnp.asarray(np.stack(rows, axis=0), dtype=jnp.bfloat16)


def kernel(x_nchw, wt, bias, gamma, beta):
    N, Cin, H, W = x_nchw.shape
    Cout = wt.shape[1]
    del bias
    f32 = jnp.float32
    HW = H * W

    w_flip = jnp.flip(wt, axis=(2, 3))
    w_stk = (jnp.transpose(w_flip, (2, 3, 0, 1))
             .reshape(9 * Cin, Cout).T.astype(jnp.bfloat16))

    masks = _boundary_masks(H, W)

    cparams = pltpu.CompilerParams(
        dimension_semantics=("parallel",),
        vmem_limit_bytes=48 << 20,
    )
    conv_flops = 2 * N * Cout * 9 * Cin * HW

    nb1, nb2 = 8, 4
    stats, xb_planes = pl.pallas_call(
        functools.partial(_stats_kernel, H=H, W=W, Cout=Cout),
        out_shape=(jax.ShapeDtypeStruct((N, Cout, _LANES), f32),
                   jax.ShapeDtypeStruct((N, Cin, HW), jnp.bfloat16)),
        grid_spec=pl.GridSpec(
            grid=(N // nb1,),
            in_specs=[
                pl.BlockSpec((nb1, Cin, H, W), lambda n: (n, 0, 0, 0)),
                pl.BlockSpec((Cout, 9 * Cin), lambda n: (0, 0)),
                pl.BlockSpec((10, HW), lambda n: (0, 0)),
            ],
            out_specs=(pl.BlockSpec((nb1, Cout, _LANES), lambda n: (n, 0, 0)),
                       pl.BlockSpec((nb1, Cin, HW), lambda n: (n, 0, 0))),
        ),
        compiler_params=cparams,
        cost_estimate=pl.CostEstimate(
            flops=conv_flops, transcendentals=0,
            bytes_accessed=4 * N * Cin * HW + 2 * N * Cin * HW
            + 4 * N * Cout * _LANES),
    )(x_nchw, w_stk, masks)

    tot = jnp.sum(stats, axis=0)
    count = N * HW
    mean = tot[:, 0] / count
    var = tot[:, 1] / count - mean * mean
    inv_std = jax.lax.rsqrt(var + _BN_EPS)
    scale_c = gamma.astype(f32) * inv_std
    shift_c = beta.astype(f32) - mean * scale_c

    w2 = jnp.concatenate(
        [w_stk.astype(f32) * scale_c.reshape(Cout, 1),
         shift_c.reshape(Cout, 1)], axis=1).astype(jnp.bfloat16)

    out_hwc = pl.pallas_call(
        functools.partial(_apply_kernel, H=H, W=W),
        out_shape=jax.ShapeDtypeStruct((N, HW, Cout), f32),
        grid_spec=pl.GridSpec(
            grid=(N // nb2,),
            in_specs=[
                pl.BlockSpec((nb2, Cin, HW), lambda n: (n, 0, 0)),
                pl.BlockSpec((Cout, 9 * Cin + 1), lambda n: (0, 0)),
                pl.BlockSpec((10, HW), lambda n: (0, 0)),
            ],
            out_specs=pl.BlockSpec((nb2, HW, Cout), lambda n: (n, 0, 0)),
        ),
        compiler_params=cparams,
        cost_estimate=pl.CostEstimate(
            flops=conv_flops + 4 * N * Cout * HW, transcendentals=0,
            bytes_accessed=2 * N * Cin * HW + 4 * N * Cout * HW),
    )(xb_planes, w2, masks)

    return jnp.transpose(out_hwc.reshape(N, H, W, Cout), (0, 3, 1, 2))


# speedup vs baseline: 2.2235x; 1.0579x over previous
import functools

import numpy as np
import jax
import jax.numpy as jnp
from jax.experimental import pallas as pl
from jax.experimental.pallas import tpu as pltpu

_BN_EPS = 1e-5
_LANES = 128


def _tap_stack(xb, m_ref, H, W):
    HW = H * W
    pieces = []
    for ky in range(3):
        for kx in range(3):
            k = ky * 3 + kx
            shift = ((ky - 1) * W + (kx - 1)) % HW
            shifted = jnp.roll(xb, shift, axis=1) if shift else xb
            if ky == 1 and kx == 1:
                pieces.append(shifted)
            else:
                pieces.append(shifted * m_ref[k:k + 1, :])
    return jnp.concatenate(pieces, axis=0)


def _stats_kernel(x_ref, w_ref, m_ref, stats_ref, xb_ref, *, H, W, Cout):
    Cin = x_ref.shape[1]
    s_tot = jnp.zeros((Cout, 1), jnp.float32)
    sq_tot = jnp.zeros((Cout, 1), jnp.float32)
    for i in range(x_ref.shape[0]):
        xb = x_ref[i].astype(jnp.bfloat16).reshape(Cin, H * W)
        xb_ref[i] = xb
        taps = _tap_stack(xb, m_ref[...], H, W)
        conv = jnp.dot(w_ref[...], taps,
                       preferred_element_type=jnp.float32)
        s_tot = s_tot + jnp.sum(conv, axis=1, keepdims=True)
        sq_tot = sq_tot + jnp.sum(conv * conv, axis=1, keepdims=True)
    pad = jnp.zeros((Cout, _LANES - 2), jnp.float32)
    acc = jnp.concatenate([s_tot, sq_tot, pad], axis=1)

    @pl.when(pl.program_id(0) == 0)
    def _init():
        stats_ref[...] = jnp.zeros_like(stats_ref)

    stats_ref[...] += acc


def _apply_kernel(xb_ref, w_ref, m_ref, o_ref, *, H, W):
    for i in range(xb_ref.shape[0]):
        xb = xb_ref[i]
        taps = _tap_stack(xb, m_ref[...], H, W)
        taps = jnp.concatenate([taps, m_ref[9:10, :]], axis=0)
        y = jnp.maximum(jnp.dot(w_ref[...], taps,
                                preferred_element_type=jnp.float32), 0.0)
        x = xb.astype(jnp.float32)
        out = y + jnp.concatenate([x, x], axis=0)
        o_ref[i] = out.T


def _boundary_masks(H, W):
    hh = np.arange(H * W) // W
    ww = np.arange(H * W) % W
    rows = []
    for ky in range(3):
        for kx in range(3):
            dy, dx = 1 - ky, 1 - kx
            ok = ((hh + dy >= 0) & (hh + dy <= H - 1) &
                  (ww + dx >= 0) & (ww + dx <= W - 1))
            rows.append(ok)
    rows.append(np.ones(H * W, dtype=bool))
    return jnp.asarray(np.stack(rows, axis=0), dtype=jnp.bfloat16)


def kernel(x_nchw, wt, bias, gamma, beta):
    N, Cin, H, W = x_nchw.shape
    Cout = wt.shape[1]
    del bias
    f32 = jnp.float32
    HW = H * W

    w_stk = (jnp.transpose(wt, (2, 3, 0, 1))
             .reshape(9 * Cin, Cout).T.astype(jnp.bfloat16))

    masks = _boundary_masks(H, W)

    cparams = pltpu.CompilerParams(
        dimension_semantics=("arbitrary",),
        vmem_limit_bytes=48 << 20,
    )
    conv_flops = 2 * N * Cout * 9 * Cin * HW

    nb1, nb2 = 8, 4
    stats, xb_planes = pl.pallas_call(
        functools.partial(_stats_kernel, H=H, W=W, Cout=Cout),
        out_shape=(jax.ShapeDtypeStruct((Cout, _LANES), f32),
                   jax.ShapeDtypeStruct((N, Cin, HW), jnp.bfloat16)),
        grid_spec=pl.GridSpec(
            grid=(N // nb1,),
            in_specs=[
                pl.BlockSpec((nb1, Cin, H, W), lambda n: (n, 0, 0, 0)),
                pl.BlockSpec((Cout, 9 * Cin), lambda n: (0, 0)),
                pl.BlockSpec((10, HW), lambda n: (0, 0)),
            ],
            out_specs=(pl.BlockSpec((Cout, _LANES), lambda n: (0, 0)),
                       pl.BlockSpec((nb1, Cin, HW), lambda n: (n, 0, 0))),
        ),
        compiler_params=cparams,
        cost_estimate=pl.CostEstimate(
            flops=conv_flops, transcendentals=0,
            bytes_accessed=4 * N * Cin * HW + 2 * N * Cin * HW
            + 4 * Cout * _LANES),
    )(x_nchw, w_stk, masks)

    count = N * HW
    mean = stats[:, 0] / count
    var = stats[:, 1] / count - mean * mean
    inv_std = jax.lax.rsqrt(var + _BN_EPS)
    scale_c = gamma.astype(f32) * inv_std
    shift_c = beta.astype(f32) - mean * scale_c

    w2 = jnp.concatenate(
        [w_stk.astype(f32) * scale_c.reshape(Cout, 1),
         shift_c.reshape(Cout, 1)], axis=1).astype(jnp.bfloat16)

    out_hwc = pl.pallas_call(
        functools.partial(_apply_kernel, H=H, W=W),
        out_shape=jax.ShapeDtypeStruct((N, HW, Cout), f32),
        grid_spec=pl.GridSpec(
            grid=(N // nb2,),
            in_specs=[
                pl.BlockSpec((nb2, Cin, HW), lambda n: (n, 0, 0)),
                pl.BlockSpec((Cout, 9 * Cin + 1), lambda n: (0, 0)),
                pl.BlockSpec((10, HW), lambda n: (0, 0)),
            ],
            out_specs=pl.BlockSpec((nb2, HW, Cout), lambda n: (n, 0, 0)),
        ),
        compiler_params=cparams,
        cost_estimate=pl.CostEstimate(
            flops=conv_flops + 4 * N * Cout * HW, transcendentals=0,
            bytes_accessed=2 * N * Cin * HW + 4 * N * Cout * HW),
    )(xb_planes, w2, masks)

    return jnp.transpose(out_hwc.reshape(N, H, W, Cout), (0, 3, 1, 2))
